# Initial kernel scaffold; baseline (speedup 1.0000x reference)
import jax
import jax.numpy as jnp
from jax.experimental import pallas as pl


def kernel(x, edge_index, batch, W0, b0, eps, W1, b1, g1, be1, W2, b2, g2, be2):
    raise NotImplementedError("write your pallas kernel here")


# speedup vs baseline: 2.5132x; 2.5132x over previous
import functools

import jax
import jax.numpy as jnp
from jax import lax
from jax.experimental import pallas as pl
from jax.experimental.pallas import tpu as pltpu
from jax.experimental.pallas import tpu_sc as plsc

_N = 10000
_E = 320000
_H = 128
_H2 = 256
_L = 4
_G = 64

_NC = 2
_NS = 16
_NW = _NC * _NS
_CHUNK = 128
_CPW = 80
_TOT_CH = _CPW * _NW
_EPAD = _TOT_CH * _CHUNK
_NPAD = _NS * 5 * _CHUNK
_RPS = _NPAD // _NS

_RB = 2000
_NBLK = _N // _RB


@functools.cache
def _make_sc_agg():
    mesh = plsc.VectorSubcoreMesh(core_axis_name="c", subcore_axis_name="s",
                                  num_cores=_NC, num_subcores=_NS)

    @functools.partial(
        pl.kernel,
        out_type=jax.ShapeDtypeStruct((_NC, _NPAD, _H), jnp.float32),
        mesh=mesh,
        scratch_types=[
            pltpu.VMEM((_CPW, _CHUNK), jnp.int32),
            pltpu.VMEM((_CPW, _CHUNK), jnp.int32),
            pltpu.VMEM((_CHUNK, _H), jnp.float32),
            pltpu.VMEM_SHARED((_NPAD, _H), jnp.float32),
            pltpu.SemaphoreType.DMA,
        ],
    )
    def agg_kernel(h_hbm, src_hbm, dst_hbm, zero_hbm, out_hbm,
                   src_v, dst_v, rows_v, agg_sh, sem):
        c = lax.axis_index("c")
        s = lax.axis_index("s")
        wid = c * _NS + s
        pltpu.sync_copy(zero_hbm, rows_v)
        row0 = s * _RPS
        for k in range(_RPS // _CHUNK):
            pltpu.sync_copy(rows_v, agg_sh.at[pl.ds(row0 + k * _CHUNK, _CHUNK)])
        plsc.subcore_barrier()
        base = wid * _CPW
        pltpu.sync_copy(src_hbm.at[pl.ds(base, _CPW)], src_v)
        pltpu.sync_copy(dst_hbm.at[pl.ds(base, _CPW)], dst_v)

        def body(j, carry):
            pltpu.async_copy(h_hbm.at[src_v.at[j]], rows_v, sem).wait()
            pltpu.sync_copy(rows_v, agg_sh.at[dst_v.at[j]], add=True)
            return carry

        lax.fori_loop(0, _CPW, body, 0)
        plsc.subcore_barrier()
        pltpu.sync_copy(agg_sh.at[pl.ds(row0, _RPS)],
                        out_hbm.at[c, pl.ds(row0, _RPS)])

    return agg_kernel


def _sc_agg(h, src2, dst2, zero_tile):
    return _make_sc_agg()(h, src2, dst2, zero_tile)


def _embed_body(x_ref, w_ref, o_ref):
    o_ref[...] = jnp.dot(x_ref[...], w_ref[...],
                         preferred_element_type=jnp.float32)


def _embed(xp, w0p):
    return pl.pallas_call(
        _embed_body,
        grid=(_NBLK,),
        in_specs=[pl.BlockSpec((_RB, 16), lambda i: (i, 0)),
                  pl.BlockSpec((16, _H), lambda i: (0, 0))],
        out_specs=pl.BlockSpec((_RB, _H), lambda i: (i, 0)),
        out_shape=jax.ShapeDtypeStruct((_N, _H), jnp.float32),
    )(xp, w0p)


def _mlp1_body(eps_ref, h_ref, part_ref, w_ref, b_ref, z1_ref, sum_ref, ss_ref):
    i = pl.program_id(0)
    z = h_ref[...] * (1.0 + eps_ref[0]) + part_ref[0] + part_ref[1]
    z1 = jnp.dot(z, w_ref[...], preferred_element_type=jnp.float32) + b_ref[...]
    z1_ref[...] = z1

    @pl.when(i == 0)
    def _():
        sum_ref[...] = jnp.zeros_like(sum_ref)
        ss_ref[...] = jnp.zeros_like(ss_ref)

    sum_ref[...] += jnp.sum(z1, axis=0)[None]
    ss_ref[...] += jnp.sum(z1 * z1, axis=0)[None]


def _mlp1(h, parts, w1, b1, eps_i):
    return pl.pallas_call(
        _mlp1_body,
        grid=(_NBLK,),
        in_specs=[
            pl.BlockSpec(memory_space=pltpu.SMEM),
            pl.BlockSpec((_RB, _H), lambda i: (i, 0)),
            pl.BlockSpec((_NC, _RB, _H), lambda i: (0, i, 0)),
            pl.BlockSpec((_H, _H2), lambda i: (0, 0)),
            pl.BlockSpec((1, _H2), lambda i: (0, 0)),
        ],
        out_specs=[
            pl.BlockSpec((_RB, _H2), lambda i: (i, 0)),
            pl.BlockSpec((1, _H2), lambda i: (0, 0)),
            pl.BlockSpec((1, _H2), lambda i: (0, 0)),
        ],
        out_shape=[
            jax.ShapeDtypeStruct((_N, _H2), jnp.float32),
            jax.ShapeDtypeStruct((1, _H2), jnp.float32),
            jax.ShapeDtypeStruct((1, _H2), jnp.float32),
        ],
    )(eps_i, h, parts, w1, b1)


def _mlp2_body(z1_ref, sum_ref, ss_ref, g_ref, be_ref, w_ref, b_ref,
               z2_ref, sum2_ref, ss2_ref):
    i = pl.program_id(0)
    mean = sum_ref[...] * (1.0 / _N)
    var = ss_ref[...] * (1.0 / _N) - mean * mean
    scale = g_ref[...] * lax.rsqrt(var + 1e-5)
    y = jnp.maximum((z1_ref[...] - mean) * scale + be_ref[...], 0.0)
    z2 = jnp.dot(y, w_ref[...], preferred_element_type=jnp.float32) + b_ref[...]
    z2_ref[...] = z2

    @pl.when(i == 0)
    def _():
        sum2_ref[...] = jnp.zeros_like(sum2_ref)
        ss2_ref[...] = jnp.zeros_like(ss2_ref)

    sum2_ref[...] += jnp.sum(z2, axis=0)[None]
    ss2_ref[...] += jnp.sum(z2 * z2, axis=0)[None]


def _mlp2(z1, s1, ss1, g1, be1, w2, b2):
    return pl.pallas_call(
        _mlp2_body,
        grid=(_NBLK,),
        in_specs=[
            pl.BlockSpec((_RB, _H2), lambda i: (i, 0)),
            pl.BlockSpec((1, _H2), lambda i: (0, 0)),
            pl.BlockSpec((1, _H2), lambda i: (0, 0)),
            pl.BlockSpec((1, _H2), lambda i: (0, 0)),
            pl.BlockSpec((1, _H2), lambda i: (0, 0)),
            pl.BlockSpec((_H2, _H), lambda i: (0, 0)),
            pl.BlockSpec((1, _H), lambda i: (0, 0)),
        ],
        out_specs=[
            pl.BlockSpec((_RB, _H), lambda i: (i, 0)),
            pl.BlockSpec((1, _H), lambda i: (0, 0)),
            pl.BlockSpec((1, _H), lambda i: (0, 0)),
        ],
        out_shape=[
            jax.ShapeDtypeStruct((_N, _H), jnp.float32),
            jax.ShapeDtypeStruct((1, _H), jnp.float32),
            jax.ShapeDtypeStruct((1, _H), jnp.float32),
        ],
    )(z1, s1, ss1, g1, be1, w2, b2)


def _mlp3_body(z2_ref, sum_ref, ss_ref, g_ref, be_ref, o_ref):
    mean = sum_ref[...] * (1.0 / _N)
    var = ss_ref[...] * (1.0 / _N) - mean * mean
    scale = g_ref[...] * lax.rsqrt(var + 1e-5)
    o_ref[...] = jnp.maximum((z2_ref[...] - mean) * scale + be_ref[...], 0.0)


def _mlp3(z2, s2, ss2, g2, be2):
    return pl.pallas_call(
        _mlp3_body,
        grid=(_NBLK,),
        in_specs=[
            pl.BlockSpec((_RB, _H), lambda i: (i, 0)),
            pl.BlockSpec((1, _H), lambda i: (0, 0)),
            pl.BlockSpec((1, _H), lambda i: (0, 0)),
            pl.BlockSpec((1, _H), lambda i: (0, 0)),
            pl.BlockSpec((1, _H), lambda i: (0, 0)),
        ],
        out_specs=pl.BlockSpec((_RB, _H), lambda i: (i, 0)),
        out_shape=jax.ShapeDtypeStruct((_N, _H), jnp.float32),
    )(z2, s2, ss2, g2, be2)


def _pool_body(h_ref, b_ref, o_ref, acc_ref, cnt_ref):
    i = pl.program_id(0)

    @pl.when(i == 0)
    def _():
        acc_ref[...] = jnp.zeros_like(acc_ref)
        cnt_ref[...] = jnp.zeros_like(cnt_ref)

    bb = b_ref[0, 0, :]
    onehot_t = (bb[None, :] ==
                lax.broadcasted_iota(jnp.int32, (_G, _RB), 0)).astype(jnp.float32)
    acc_ref[...] += jnp.dot(onehot_t, h_ref[...],
                            preferred_element_type=jnp.float32)
    cnt_ref[...] += jnp.broadcast_to(jnp.sum(onehot_t, axis=1)[:, None],
                                     (_G, _H))

    @pl.when(i == _NBLK - 1)
    def _():
        o_ref[...] = acc_ref[...] / jnp.maximum(cnt_ref[...], 1.0)


def _pool(h, batch3):
    return pl.pallas_call(
        _pool_body,
        grid=(_NBLK,),
        in_specs=[
            pl.BlockSpec((_RB, _H), lambda i: (i, 0)),
            pl.BlockSpec((1, 1, _RB), lambda i: (i, 0, 0)),
        ],
        out_specs=pl.BlockSpec((_G, _H), lambda i: (0, 0)),
        out_shape=jax.ShapeDtypeStruct((_G, _H), jnp.float32),
        scratch_shapes=[pltpu.VMEM((_G, _H), jnp.float32),
                        pltpu.VMEM((_G, _H), jnp.float32)],
    )(h, batch3)


def kernel(x, edge_index, batch, W0, b0, eps, W1, b1, g1, be1, W2, b2, g2, be2):
    ones_col = jnp.ones((_N, 1), jnp.float32)
    xp = jnp.concatenate(
        [x, ones_col, jnp.zeros((_N, 16 - 9 - 1), jnp.float32)], axis=1)
    w0p = jnp.concatenate(
        [W0, b0[None, :], jnp.zeros((16 - 9 - 1, _H), jnp.float32)], axis=0)

    pad = _EPAD - _E
    src2 = jnp.concatenate(
        [edge_index[0], jnp.zeros((pad,), jnp.int32)]).reshape(_TOT_CH, _CHUNK)
    dst2 = jnp.concatenate(
        [edge_index[1], jnp.full((pad,), _N, jnp.int32)]).reshape(_TOT_CH, _CHUNK)
    zero_tile = jnp.zeros((_CHUNK, _H), jnp.float32)
    batch3 = batch.reshape(_NBLK, 1, _RB)

    h = _embed(xp, w0p)
    for i in range(_L):
        parts = _sc_agg(h, src2, dst2, zero_tile)
        z1, s1, ss1 = _mlp1(h, parts, W1[i], b1[i][None], eps[i][None])
        z2, s2, ss2 = _mlp2(z1, s1, ss1, g1[i][None], be1[i][None],
                            W2[i], b2[i][None])
        h = _mlp3(z2, s2, ss2, g2[i][None], be2[i][None])
    return _pool(h, batch3)
